# baseline (speedup 1.0000x reference)
import jax
import jax.numpy as jnp
from jax import lax
from jax.experimental import pallas as pl
from jax.experimental.pallas import tpu as pltpu
from jax.experimental.pallas import tpu_sc as plsc

NUM_CLASSES = 1000
D = 512
N = 100000
B = 32
NB = N // B
NW = 32
ALPHA = 1.0
TAU_LAMBDA = 1.5
SC_W = 16


def _rsqrt16(x):
    i = plsc.bitcast(x, jnp.int32)
    y = plsc.bitcast(jnp.int32(0x5F3759DF) - (i >> 1), jnp.float32)
    for _ in range(3):
        y = y * (1.5 - 0.5 * x * y * y)
    return y


_GDN = lax.GatherDimensionNumbers(offset_dims=(), collapsed_slice_dims=(0,),
                                  start_index_map=(0,))


def _perm(v, idx):
    return lax.gather(v, idx[:, None], _GDN, (1,),
                      mode=lax.GatherScatterMode.PROMISE_IN_BOUNDS)


def _lane_sum(v, rots):
    for r in rots:
        v = v + _perm(v, r)
    return v


def _compute_batch(p_ref, a_ref, scal_ref, iota16, rots):
    z = jnp.zeros((16,), jnp.float32)
    for g in range(B // 16):

        def row_body(r, carry):
            s_v, qp_v, qa_v = carry
            row = g * 16 + r
            sch = [z, z, z, z]
            qph = [z, z, z, z]
            qah = [z, z, z, z]
            for k in range(D // 16):
                pc = p_ref[row, pl.ds(16 * k, 16)]
                ac = a_ref[row, pl.ds(16 * k, 16)]
                j = k & 3
                sch[j] = sch[j] + pc * ac
                qph[j] = qph[j] + pc * pc
                qah[j] = qah[j] + ac * ac
            s = _lane_sum((sch[0] + sch[1]) + (sch[2] + sch[3]), rots)
            qp = _lane_sum((qph[0] + qph[1]) + (qph[2] + qph[3]), rots)
            qa = _lane_sum((qah[0] + qah[1]) + (qah[2] + qah[3]), rots)
            mask = iota16 == r
            return (jnp.where(mask, s, s_v), jnp.where(mask, qp, qp_v),
                    jnp.where(mask, qa, qa_v))

        s_v, qp_v, qa_v = lax.fori_loop(0, 16, row_body, (z, z, z), unroll=2)
        prod = qp_v * qa_v
        sq = prod * _rsqrt16(prod)
        cos = s_v / jnp.maximum(sq, 1e-8)
        rows = iota16 + g * 16
        plsc.store_scatter(scal_ref, [rows, jnp.full((16,), 1, jnp.int32)],
                           cos)
        plsc.store_scatter(scal_ref, [rows, jnp.full((16,), 2, jnp.int32)],
                           cos * cos)


def _sc_body(patches, labels, text, zbig, zsmall, out_sums, out_scal,
             p0, p1, a0, a1, lab0, lab1, scal, sh_sums, sh_scal,
             sem_p0, sem_p1, sem_a0, sem_a1, sem_l0, sem_l1):
    c = lax.axis_index("c")
    s = lax.axis_index("s")
    wid = s * 2 + c

    @pl.when(s == 0)
    def _():
        pltpu.sync_copy(zbig, sh_sums)
        pltpu.sync_copy(zsmall, sh_scal)

    iota16 = lax.iota(jnp.int32, 16)
    one_row = jnp.where(iota16 == 0, 1.0, 0.0).astype(jnp.float32)
    for r in range(B):
        scal[r, :] = one_row
    rots = [(iota16 + sh) & 15 for sh in (1, 2, 4, 8)]
    plsc.subcore_barrier()

    n_mine = (NB - 1 - wid) // NW + 1

    def _issue_pl(kc, lab_b, p_b, sem_l_b, sem_p_b):
        base = (wid + kc * NW) * B
        pltpu.async_copy(labels.at[pl.ds(base, B)], lab_b, sem_l_b)
        pltpu.async_copy(patches.at[pl.ds(base, B)], p_b, sem_p_b)

    def _half(kc, p_b, a_b, lab_b, sem_p_b, sem_a_b,
              a_o, lab_o, sem_a_o, sem_l_o, sem_l_b):
        @pl.when(kc < n_mine)
        def _():
            pltpu.make_async_copy(patches.at[pl.ds(0, B)], p_b, sem_p_b).wait()
            pltpu.make_async_copy(patches.at[pl.ds(0, B)], a_b, sem_a_b).wait()

            @pl.when(kc + 1 < n_mine)
            def _():
                pltpu.make_async_copy(labels.at[pl.ds(0, B)], lab_o,
                                      sem_l_o).wait()
                pltpu.async_copy(text.at[lab_o], a_o, sem_a_o)

            _compute_batch(p_b, a_b, scal, iota16, rots)


            @pl.when(kc + 2 < n_mine)
            def _():
                _issue_pl(kc + 2, lab_b, p_b, sem_l_b, sem_p_b)

    base0 = wid * B
    pltpu.sync_copy(labels.at[pl.ds(base0, B)], lab0)
    pltpu.async_copy(patches.at[pl.ds(base0, B)], p0, sem_p0)
    pltpu.async_copy(text.at[lab0], a0, sem_a0)

    @pl.when(1 < n_mine)
    def _():
        _issue_pl(1, lab1, p1, sem_l1, sem_p1)

    def pair_body(kk, carry):
        _half(2 * kk, p0, a0, lab0, sem_p0, sem_a0,
              a1, lab1, sem_a1, sem_l1, sem_l0)
        _half(2 * kk + 1, p1, a1, lab1, sem_p1, sem_a1,
              a0, lab0, sem_a0, sem_l0, sem_l1)
        return carry

    lax.fori_loop(0, (n_mine + 1) // 2, pair_body, 0)

    plsc.subcore_barrier()

    @pl.when(s == 0)
    def _():
        pltpu.sync_copy(sh_sums, out_sums.at[c])
        pltpu.sync_copy(sh_scal, out_scal.at[c])


def _finish_body(ps_ref, sc_ref, t_ref, u_ref, tau_ref):
    cs = ps_ref[0] + ps_ref[1]
    scal = sc_ref[0] + sc_ref[1]
    cnt = scal[:, 0:1]
    scos = scal[:, 1:2]
    scos2 = scal[:, 2:3]

    sum_d = cnt - scos
    sum_d2 = cnt - 2.0 * scos + scos2
    safe = jnp.maximum(cnt, 1.0)
    mu = sum_d / safe
    var = (sum_d2 - cnt * mu * mu) / jnp.maximum(cnt - 1.0, 1.0)
    std = jnp.sqrt(jnp.maximum(var, 0.0))
    tau = jnp.where(std > 0.0, mu + TAU_LAMBDA * std, mu + 0.1)

    proto = cs / safe
    pn = jnp.sqrt(jnp.sum(proto * proto, axis=-1, keepdims=True))
    proto = proto / jnp.maximum(pn, 1e-12)
    un = t_ref[...] + ALPHA * proto
    unn = jnp.sqrt(jnp.sum(un * un, axis=-1, keepdims=True))
    u_ref[...] = un / jnp.maximum(unn, 1e-12)
    tau_ref[...] = tau


_sc_kernel = pl.kernel(
    _sc_body,
    out_type=(
        jax.ShapeDtypeStruct((2, NUM_CLASSES, D), jnp.float32),
        jax.ShapeDtypeStruct((2, NUM_CLASSES, SC_W), jnp.float32),
    ),
    mesh=plsc.VectorSubcoreMesh(core_axis_name="c", subcore_axis_name="s"),
    compiler_params=pltpu.CompilerParams(use_tc_tiling_on_sc=False,
                                         needs_layout_passes=False),
    scratch_types=[
        pltpu.VMEM((B, D), jnp.float32),
        pltpu.VMEM((B, D), jnp.float32),
        pltpu.VMEM((B, D), jnp.float32),
        pltpu.VMEM((B, D), jnp.float32),
        pltpu.VMEM((B,), jnp.int32),
        pltpu.VMEM((B,), jnp.int32),
        pltpu.VMEM((B, SC_W), jnp.float32),
        pltpu.VMEM_SHARED((NUM_CLASSES, D), jnp.float32),
        pltpu.VMEM_SHARED((NUM_CLASSES, SC_W), jnp.float32),
        pltpu.SemaphoreType.DMA,
        pltpu.SemaphoreType.DMA,
        pltpu.SemaphoreType.DMA,
        pltpu.SemaphoreType.DMA,
        pltpu.SemaphoreType.DMA,
        pltpu.SemaphoreType.DMA,
    ],
)

_tc_finish = pl.pallas_call(
    _finish_body,
    out_shape=(
        jax.ShapeDtypeStruct((NUM_CLASSES, D), jnp.float32),
        jax.ShapeDtypeStruct((NUM_CLASSES, 1), jnp.float32),
    ),
)


def kernel(support_patches, support_labels, text_features):
    labels_i32 = support_labels.astype(jnp.int32)
    zbig = jnp.zeros((NUM_CLASSES, D), jnp.float32)
    zsmall = jnp.zeros((NUM_CLASSES, SC_W), jnp.float32)
    psums, pscal = _sc_kernel(support_patches, labels_i32, text_features,
                              zbig, zsmall)
    unified, tau = _tc_finish(psums, pscal, text_features)
    return unified, tau[:, 0]


# speedup vs baseline: 1.1512x; 1.1512x over previous
import jax
import jax.numpy as jnp
from jax import lax
from jax.experimental import pallas as pl
from jax.experimental.pallas import tpu as pltpu
from jax.experimental.pallas import tpu_sc as plsc

NUM_CLASSES = 1000
D = 512
N = 100000
B = 16
NB = N // B
NW = 32
ALPHA = 1.0
TAU_LAMBDA = 1.5
SC_W = 16


def _rsqrt16(x):
    i = plsc.bitcast(x, jnp.int32)
    y = plsc.bitcast(jnp.int32(0x5F3759DF) - (i >> 1), jnp.float32)
    for _ in range(3):
        y = y * (1.5 - 0.5 * x * y * y)
    return y


_GDN = lax.GatherDimensionNumbers(offset_dims=(), collapsed_slice_dims=(0,),
                                  start_index_map=(0,))


def _perm(v, idx):
    return lax.gather(v, idx[:, None], _GDN, (1,),
                      mode=lax.GatherScatterMode.PROMISE_IN_BOUNDS)


def _lane_sum(v, rots):
    for r in rots:
        v = v + _perm(v, r)
    return v


def _compute_batch(p_ref, a_ref, scal_ref, iota16, rots):
    def row_body(row, carry):
        pc0 = p_ref[row, pl.ds(0, 16)]
        ac0 = a_ref[row, pl.ds(0, 16)]
        pc1 = p_ref[row, pl.ds(16, 16)]
        ac1 = a_ref[row, pl.ds(16, 16)]
        sch = [pc0 * ac0, pc1 * ac1]
        qph = [pc0 * pc0, pc1 * pc1]
        qah = [ac0 * ac0, ac1 * ac1]
        for k in range(2, D // 16):
            pc = p_ref[row, pl.ds(16 * k, 16)]
            ac = a_ref[row, pl.ds(16 * k, 16)]
            j = k & 1
            sch[j] = sch[j] + pc * ac
            qph[j] = qph[j] + pc * pc
            qah[j] = qah[j] + ac * ac
        s = _lane_sum(sch[0] + sch[1], rots)
        qp = _lane_sum(qph[0] + qph[1], rots)
        qa = _lane_sum(qah[0] + qah[1], rots)
        prod = qp * qa
        sq = prod * _rsqrt16(prod)
        cos = s / jnp.maximum(sq, 1e-8)
        v = jnp.where(iota16 == 0, cos, cos * cos)
        plsc.store_scatter(scal_ref,
                           [jnp.full((16,), 0, jnp.int32) + row,
                            jnp.minimum(iota16 + 1, 2)],
                           v, mask=iota16 < 2)
        return carry

    lax.fori_loop(0, B, row_body, 0)


def _sc_body(patches, labels, text, out_sums, out_scal,
             p_b, a_b, lab_b, scal_b, sh_sums, sh_scal,
             sem_p, sem_a, sem_l, sem_sp, sem_ss):
    c = lax.axis_index("c")
    s = lax.axis_index("s")
    wid = s * 2 + c

    iota16 = lax.iota(jnp.int32, 16)
    z16 = jnp.zeros((16,), jnp.float32)

    for r in range(8):
        for k in range(D // 16):
            p_b[0][r, pl.ds(16 * k, 16)] = z16
    for r in range(B):
        scal_b[0][r, :] = z16
    nblk = NUM_CLASSES // 8
    for m in range(8):
        blk = s + m * 16

        @pl.when(blk < nblk)
        def _():
            pltpu.sync_copy(p_b[0].at[pl.ds(0, 8)],
                            sh_sums.at[pl.ds(blk * 8, 8)])

    @pl.when(s == 0)
    def _():
        for m in range(NUM_CLASSES // B):
            pltpu.sync_copy(scal_b[0], sh_scal.at[pl.ds(m * B, B)])
        pltpu.sync_copy(scal_b[0].at[pl.ds(0, 8)],
                        sh_scal.at[pl.ds((NUM_CLASSES // B) * B, 8)])

    one_row = jnp.where(iota16 == 0, 1.0, 0.0).astype(jnp.float32)
    for buf in range(3):
        for r in range(B):
            scal_b[buf][r, :] = one_row
    rots = [(iota16 + sh) & 15 for sh in (1, 2, 4, 8)]
    plsc.subcore_barrier()

    n_mine = (NB - 1 - wid) // NW + 1

    def _issue_pl(kc, i):
        base = (wid + kc * NW) * B
        pltpu.async_copy(labels.at[pl.ds(base, B)], lab_b[i], sem_l[i])
        pltpu.async_copy(patches.at[pl.ds(base, B)], p_b[i], sem_p[i])

    def _drain(dummy_src, dst, sem):
        pltpu.make_async_copy(dummy_src, dst, sem).wait()

    def _half(kc, i):
        inext = (i + 1) % 3
        iprev = (i + 2) % 3

        @pl.when(kc < n_mine)
        def _():
            _drain(patches.at[pl.ds(0, B)], p_b[i], sem_p[i])
            _drain(patches.at[pl.ds(0, B)], a_b[i], sem_a[i])

            @pl.when(kc + 1 < n_mine)
            def _():
                _drain(labels.at[pl.ds(0, B)], lab_b[inext], sem_l[inext])
                pltpu.async_copy(text.at[lab_b[inext]], a_b[inext],
                                 sem_a[inext])

            _compute_batch(p_b[i], a_b[i], scal_b[i], iota16, rots)

            pltpu.sync_copy(p_b[i], sh_sums.at[lab_b[i]], add=True)
            pltpu.sync_copy(scal_b[i], sh_scal.at[lab_b[i]], add=True)

            @pl.when(kc + 3 < n_mine)
            def _():
                _issue_pl(kc + 3, i)

    base0 = wid * B
    pltpu.sync_copy(labels.at[pl.ds(base0, B)], lab_b[0])
    pltpu.async_copy(patches.at[pl.ds(base0, B)], p_b[0], sem_p[0])
    pltpu.async_copy(text.at[lab_b[0]], a_b[0], sem_a[0])

    @pl.when(1 < n_mine)
    def _():
        _issue_pl(1, 1)

    @pl.when(2 < n_mine)
    def _():
        _issue_pl(2, 2)

    def trip_body(kk, carry):
        _half(3 * kk, 0)
        _half(3 * kk + 1, 1)
        _half(3 * kk + 2, 2)
        return carry

    lax.fori_loop(0, (n_mine + 2) // 3, trip_body, 0)

    plsc.subcore_barrier()

    @pl.when(s == 0)
    def _():
        pltpu.sync_copy(sh_sums, out_sums.at[c])
        pltpu.sync_copy(sh_scal, out_scal.at[c])


def _finish_body(ps_ref, sc_ref, t_ref, u_ref, tau_ref):
    cs = ps_ref[0] + ps_ref[1]
    scal = sc_ref[0] + sc_ref[1]
    cnt = scal[:, 0:1]
    scos = scal[:, 1:2]
    scos2 = scal[:, 2:3]

    sum_d = cnt - scos
    sum_d2 = cnt - 2.0 * scos + scos2
    safe = jnp.maximum(cnt, 1.0)
    mu = sum_d / safe
    var = (sum_d2 - cnt * mu * mu) / jnp.maximum(cnt - 1.0, 1.0)
    std = jnp.sqrt(jnp.maximum(var, 0.0))
    tau = jnp.where(std > 0.0, mu + TAU_LAMBDA * std, mu + 0.1)

    proto = cs / safe
    pn = jnp.sqrt(jnp.sum(proto * proto, axis=-1, keepdims=True))
    proto = proto / jnp.maximum(pn, 1e-12)
    un = t_ref[...] + ALPHA * proto
    unn = jnp.sqrt(jnp.sum(un * un, axis=-1, keepdims=True))
    u_ref[...] = un / jnp.maximum(unn, 1e-12)
    tau_ref[...] = tau


def _sc_entry(patches, labels, text, out_sums, out_scal,
              p0, p1, p2, a0, a1, a2b, l0, l1, l2, s0, s1, s2,
              sh_sums, sh_scal,
              sp0, sp1, sp2, sa0, sa1, sa2, sl0, sl1, sl2,
              ssp0, ssp1, ssp2, sss0, sss1, sss2):
    _sc_body(patches, labels, text, out_sums, out_scal,
             [p0, p1, p2], [a0, a1, a2b], [l0, l1, l2], [s0, s1, s2],
             sh_sums, sh_scal,
             [sp0, sp1, sp2], [sa0, sa1, sa2], [sl0, sl1, sl2],
             [ssp0, ssp1, ssp2], [sss0, sss1, sss2])


_sc_kernel = pl.kernel(
    _sc_entry,
    out_type=(
        jax.ShapeDtypeStruct((2, NUM_CLASSES, D), jnp.float32),
        jax.ShapeDtypeStruct((2, NUM_CLASSES, SC_W), jnp.float32),
    ),
    mesh=plsc.VectorSubcoreMesh(core_axis_name="c", subcore_axis_name="s"),
    compiler_params=pltpu.CompilerParams(use_tc_tiling_on_sc=False,
                                         needs_layout_passes=False),
    scratch_types=(
        [pltpu.VMEM((B, D), jnp.float32) for _ in range(3)]
        + [pltpu.VMEM((B, D), jnp.float32) for _ in range(3)]
        + [pltpu.VMEM((B,), jnp.int32) for _ in range(3)]
        + [pltpu.VMEM((B, SC_W), jnp.float32) for _ in range(3)]
        + [pltpu.VMEM_SHARED((NUM_CLASSES, D), jnp.float32),
           pltpu.VMEM_SHARED((NUM_CLASSES, SC_W), jnp.float32)]
        + [pltpu.SemaphoreType.DMA] * 15
    ),
)

_tc_finish = pl.pallas_call(
    _finish_body,
    out_shape=(
        jax.ShapeDtypeStruct((NUM_CLASSES, D), jnp.float32),
        jax.ShapeDtypeStruct((NUM_CLASSES, 1), jnp.float32),
    ),
)


def kernel(support_patches, support_labels, text_features):
    labels_i32 = support_labels.astype(jnp.int32)
    psums, pscal = _sc_kernel(support_patches, labels_i32, text_features)
    unified, tau = _tc_finish(psums, pscal, text_features)
    return unified, tau[:, 0]


# speedup vs baseline: 1.1614x; 1.0088x over previous
import jax
import jax.numpy as jnp
from jax import lax
from jax.experimental import pallas as pl
from jax.experimental.pallas import tpu as pltpu
from jax.experimental.pallas import tpu_sc as plsc

NUM_CLASSES = 1000
D = 512
N = 100000
B = 16
NB = N // B
NW = 32
ALPHA = 1.0
TAU_LAMBDA = 1.5
SC_W = 16


def _rsqrt16(x):
    i = plsc.bitcast(x, jnp.int32)
    y = plsc.bitcast(jnp.int32(0x5F3759DF) - (i >> 1), jnp.float32)
    for _ in range(3):
        y = y * (1.5 - 0.5 * x * y * y)
    return y


_GDN = lax.GatherDimensionNumbers(offset_dims=(), collapsed_slice_dims=(0,),
                                  start_index_map=(0,))


def _perm(v, idx):
    return lax.gather(v, idx[:, None], _GDN, (1,),
                      mode=lax.GatherScatterMode.PROMISE_IN_BOUNDS)


def _lane_sum(v, rots):
    for r in rots:
        v = v + _perm(v, r)
    return v


def _compute_batch(p_ref, a_ref, scal_ref, iota16, rots):
    def row_body(row, carry):
        pc0 = p_ref[row, pl.ds(0, 16)]
        ac0 = a_ref[row, pl.ds(0, 16)]
        pc1 = p_ref[row, pl.ds(16, 16)]
        ac1 = a_ref[row, pl.ds(16, 16)]
        sch = [pc0 * ac0, pc1 * ac1]
        qph = [pc0 * pc0, pc1 * pc1]
        qah = [ac0 * ac0, ac1 * ac1]
        for k in range(2, D // 16):
            pc = p_ref[row, pl.ds(16 * k, 16)]
            ac = a_ref[row, pl.ds(16 * k, 16)]
            j = k & 1
            sch[j] = sch[j] + pc * ac
            qph[j] = qph[j] + pc * pc
            qah[j] = qah[j] + ac * ac
        s = _lane_sum(sch[0] + sch[1], rots)
        qp = _lane_sum(qph[0] + qph[1], rots)
        qa = _lane_sum(qah[0] + qah[1], rots)
        prod = qp * qa
        sq = prod * _rsqrt16(prod)
        cos = s / jnp.maximum(sq, 1e-8)
        v = jnp.where(iota16 == 0, cos, cos * cos)
        plsc.store_scatter(scal_ref,
                           [jnp.full((16,), 0, jnp.int32) + row,
                            jnp.minimum(iota16 + 1, 2)],
                           v, mask=iota16 < 2)
        return carry

    lax.fori_loop(0, B, row_body, 0)


def _sc_body(patches, labels, text, out_sums, out_scal,
             p_b, a_b, lab_b, scal_b, sh_sums, sh_scal,
             sem_p, sem_a, sem_l, sem_sp, sem_ss):
    c = lax.axis_index("c")
    s = lax.axis_index("s")
    wid = s * 2 + c

    iota16 = lax.iota(jnp.int32, 16)
    z16 = jnp.zeros((16,), jnp.float32)

    for r in range(8):
        for k in range(D // 16):
            p_b[0][r, pl.ds(16 * k, 16)] = z16
    for r in range(B):
        scal_b[0][r, :] = z16
    nblk = NUM_CLASSES // 8
    for m in range(8):
        blk = s + m * 16

        @pl.when(blk < nblk)
        def _():
            pltpu.sync_copy(p_b[0].at[pl.ds(0, 8)],
                            sh_sums.at[pl.ds(blk * 8, 8)])

    @pl.when(s == 0)
    def _():
        for m in range(NUM_CLASSES // B):
            pltpu.sync_copy(scal_b[0], sh_scal.at[pl.ds(m * B, B)])
        pltpu.sync_copy(scal_b[0].at[pl.ds(0, 8)],
                        sh_scal.at[pl.ds((NUM_CLASSES // B) * B, 8)])

    one_row = jnp.where(iota16 == 0, 1.0, 0.0).astype(jnp.float32)
    for buf in range(3):
        for r in range(B):
            scal_b[buf][r, :] = one_row
    rots = [(iota16 + sh) & 15 for sh in (1, 2, 4, 8)]
    plsc.subcore_barrier()

    n_mine = (NB - 1 - wid) // NW + 1

    def _issue_pl(kc, i):
        base = (wid + kc * NW) * B
        pltpu.async_copy(labels.at[pl.ds(base, B)], lab_b[i], sem_l[i])
        pltpu.async_copy(patches.at[pl.ds(base, B)], p_b[i], sem_p[i])

    def _drain(dummy_src, dst, sem):
        pltpu.make_async_copy(dummy_src, dst, sem).wait()

    def _half(kc, i):
        inext = (i + 1) % 3
        iprev = (i + 2) % 3

        @pl.when(kc < n_mine)
        def _():
            _drain(patches.at[pl.ds(0, B)], p_b[i], sem_p[i])
            _drain(patches.at[pl.ds(0, B)], a_b[i], sem_a[i])

            @pl.when(kc + 1 < n_mine)
            def _():
                _drain(labels.at[pl.ds(0, B)], lab_b[inext], sem_l[inext])
                pltpu.async_copy(text.at[lab_b[inext]], a_b[inext],
                                 sem_a[inext])

            _compute_batch(p_b[i], a_b[i], scal_b[i], iota16, rots)

            @pl.when(kc >= 1)
            def _():
                _drain(p_b[iprev], sh_sums.at[lab_b[iprev]], sem_sp[iprev])
                _drain(scal_b[iprev], sh_scal.at[lab_b[iprev]],
                       sem_ss[iprev])

            pltpu.async_copy(p_b[i], sh_sums.at[lab_b[i]], sem_sp[i],
                             add=True)
            pltpu.async_copy(scal_b[i], sh_scal.at[lab_b[i]], sem_ss[i],
                             add=True)

            @pl.when(kc + 2 < n_mine)
            def _():
                _issue_pl(kc + 2, iprev)

    base0 = wid * B
    pltpu.sync_copy(labels.at[pl.ds(base0, B)], lab_b[0])
    pltpu.async_copy(patches.at[pl.ds(base0, B)], p_b[0], sem_p[0])
    pltpu.async_copy(text.at[lab_b[0]], a_b[0], sem_a[0])

    @pl.when(1 < n_mine)
    def _():
        _issue_pl(1, 1)

    def trip_body(kk, carry):
        _half(3 * kk, 0)
        _half(3 * kk + 1, 1)
        _half(3 * kk + 2, 2)
        return carry

    lax.fori_loop(0, (n_mine + 2) // 3, trip_body, 0)

    for j in range(3):
        @pl.when((n_mine - 1) % 3 == j)
        def _():
            _drain(p_b[j], sh_sums.at[lab_b[j]], sem_sp[j])
            _drain(scal_b[j], sh_scal.at[lab_b[j]], sem_ss[j])

    plsc.subcore_barrier()

    @pl.when(s == 0)
    def _():
        pltpu.sync_copy(sh_sums, out_sums.at[c])
        pltpu.sync_copy(sh_scal, out_scal.at[c])


def _finish_body(ps_ref, sc_ref, t_ref, u_ref, tau_ref):
    cs = ps_ref[0] + ps_ref[1]
    scal = sc_ref[0] + sc_ref[1]
    cnt = scal[:, 0:1]
    scos = scal[:, 1:2]
    scos2 = scal[:, 2:3]

    sum_d = cnt - scos
    sum_d2 = cnt - 2.0 * scos + scos2
    safe = jnp.maximum(cnt, 1.0)
    mu = sum_d / safe
    var = (sum_d2 - cnt * mu * mu) / jnp.maximum(cnt - 1.0, 1.0)
    std = jnp.sqrt(jnp.maximum(var, 0.0))
    tau = jnp.where(std > 0.0, mu + TAU_LAMBDA * std, mu + 0.1)

    proto = cs / safe
    pn = jnp.sqrt(jnp.sum(proto * proto, axis=-1, keepdims=True))
    proto = proto / jnp.maximum(pn, 1e-12)
    un = t_ref[...] + ALPHA * proto
    unn = jnp.sqrt(jnp.sum(un * un, axis=-1, keepdims=True))
    u_ref[...] = un / jnp.maximum(unn, 1e-12)
    tau_ref[...] = tau


def _sc_entry(patches, labels, text, out_sums, out_scal,
              p0, p1, p2, a0, a1, a2b, l0, l1, l2, s0, s1, s2,
              sh_sums, sh_scal,
              sp0, sp1, sp2, sa0, sa1, sa2, sl0, sl1, sl2,
              ssp0, ssp1, ssp2, sss0, sss1, sss2):
    _sc_body(patches, labels, text, out_sums, out_scal,
             [p0, p1, p2], [a0, a1, a2b], [l0, l1, l2], [s0, s1, s2],
             sh_sums, sh_scal,
             [sp0, sp1, sp2], [sa0, sa1, sa2], [sl0, sl1, sl2],
             [ssp0, ssp1, ssp2], [sss0, sss1, sss2])


_sc_kernel = pl.kernel(
    _sc_entry,
    out_type=(
        jax.ShapeDtypeStruct((2, NUM_CLASSES, D), jnp.float32),
        jax.ShapeDtypeStruct((2, NUM_CLASSES, SC_W), jnp.float32),
    ),
    mesh=plsc.VectorSubcoreMesh(core_axis_name="c", subcore_axis_name="s"),
    compiler_params=pltpu.CompilerParams(use_tc_tiling_on_sc=False,
                                         needs_layout_passes=False),
    scratch_types=(
        [pltpu.VMEM((B, D), jnp.float32) for _ in range(3)]
        + [pltpu.VMEM((B, D), jnp.float32) for _ in range(3)]
        + [pltpu.VMEM((B,), jnp.int32) for _ in range(3)]
        + [pltpu.VMEM((B, SC_W), jnp.float32) for _ in range(3)]
        + [pltpu.VMEM_SHARED((NUM_CLASSES, D), jnp.float32),
           pltpu.VMEM_SHARED((NUM_CLASSES, SC_W), jnp.float32)]
        + [pltpu.SemaphoreType.DMA] * 15
    ),
)

_tc_finish = pl.pallas_call(
    _finish_body,
    out_shape=(
        jax.ShapeDtypeStruct((NUM_CLASSES, D), jnp.float32),
        jax.ShapeDtypeStruct((NUM_CLASSES, 1), jnp.float32),
    ),
)


def kernel(support_patches, support_labels, text_features):
    labels_i32 = support_labels.astype(jnp.int32)
    psums, pscal = _sc_kernel(support_patches, labels_i32, text_features)
    unified, tau = _tc_finish(psums, pscal, text_features)
    return unified, tau[:, 0]
